# baseline (speedup 1.0000x reference)
import functools
import math

import jax
import jax.numpy as jnp
import numpy as np
from jax import lax
from jax.experimental import pallas as pl
from jax.experimental.pallas import tpu as pltpu
from jax.experimental.pallas import tpu_sc as plsc

T = 524288
L = 16
F = 2
N_MIN = 16
N_MAX = 2048
NUM_POINTS = 262144

NC = 2
NS = 16
NW = NC * NS
LANES = 16

PW = NUM_POINTS // NW
B = 512
NBLK = PW // B
BG = B // LANES
HALF = B * 8

_GROWTH = math.exp((math.log(N_MAX) - math.log(N_MIN)) / (L - 1))
SCALES = [float(math.floor(N_MIN * (_GROWTH ** l))) for l in range(L)]
P2 = -1640531535
P3 = 805459861
MASK = T - 1


def _body(xs_hbm, ys_hbm, zs_hbm, table_hbm, out_hbm,
          xv, yv, zv, fxv, fyv, fzv, idxv, gathv, outv, sem):
    wid = lax.axis_index("s") * NC + lax.axis_index("c")
    iota = lax.iota(jnp.int32, LANES)

    def blk_body(blk, carry):
        base = wid * PW + blk * B
        pltpu.sync_copy(xs_hbm.at[pl.ds(base, B)], xv)
        pltpu.sync_copy(ys_hbm.at[pl.ds(base, B)], yv)
        pltpu.sync_copy(zs_hbm.at[pl.ds(base, B)], zv)

        for lvl in range(L):
            scale = np.float32(SCALES[lvl])
            lvl_t2 = np.int32(lvl * T * 2)

            def idx_body(g, c0, lvl_t2=lvl_t2, scale=scale):
                sl = pl.ds(g * LANES, LANES)
                px = xv[sl] * scale
                py = yv[sl] * scale
                pz = zv[sl] * scale
                ix = px.astype(jnp.int32)
                iy = py.astype(jnp.int32)
                iz = pz.astype(jnp.int32)
                fxv[sl] = px - ix.astype(jnp.float32)
                fyv[sl] = py - iy.astype(jnp.float32)
                fzv[sl] = pz - iz.astype(jnp.float32)
                yp = iy * P2
                zp = iz * P3
                hy1 = yp + P2
                hz1 = zp + P3
                a0 = (((ix & MASK) << 1) ^ lvl_t2)
                a1 = ((((ix + 1) & MASK) << 1) ^ lvl_t2)
                b00 = ((yp ^ zp) & MASK) << 1
                b01 = ((yp ^ hz1) & MASK) << 1
                b10 = ((hy1 ^ zp) & MASK) << 1
                b11 = ((hy1 ^ hz1) & MASK) << 1
                rb = g * (8 * LANES)
                e0 = a0 ^ b00
                e1 = a0 ^ b01
                e2 = a0 ^ b10
                e3 = a0 ^ b11
                e4 = a1 ^ b00
                e5 = a1 ^ b01
                e6 = a1 ^ b10
                e7 = a1 ^ b11
                idxv[pl.ds(rb + 0 * LANES, LANES)] = e0
                idxv[pl.ds(rb + 1 * LANES, LANES)] = e1
                idxv[pl.ds(rb + 2 * LANES, LANES)] = e2
                idxv[pl.ds(rb + 3 * LANES, LANES)] = e3
                idxv[pl.ds(rb + 4 * LANES, LANES)] = e4
                idxv[pl.ds(rb + 5 * LANES, LANES)] = e5
                idxv[pl.ds(rb + 6 * LANES, LANES)] = e6
                idxv[pl.ds(rb + 7 * LANES, LANES)] = e7
                hb = HALF + rb
                idxv[pl.ds(hb + 0 * LANES, LANES)] = e0 + 1
                idxv[pl.ds(hb + 1 * LANES, LANES)] = e1 + 1
                idxv[pl.ds(hb + 2 * LANES, LANES)] = e2 + 1
                idxv[pl.ds(hb + 3 * LANES, LANES)] = e3 + 1
                idxv[pl.ds(hb + 4 * LANES, LANES)] = e4 + 1
                idxv[pl.ds(hb + 5 * LANES, LANES)] = e5 + 1
                idxv[pl.ds(hb + 6 * LANES, LANES)] = e6 + 1
                idxv[pl.ds(hb + 7 * LANES, LANES)] = e7 + 1
                return c0

            lax.fori_loop(0, BG, idx_body, 0)

            pltpu.async_copy(table_hbm.at[idxv], gathv, sem).wait()

            def comb_body(g, c0, lvl=lvl):
                sl = pl.ds(g * LANES, LANES)
                fx = fxv[sl]
                fy = fyv[sl]
                fz = fzv[sl]
                gx = 1.0 - fx
                gy = 1.0 - fy
                gz = 1.0 - fz
                w00 = gx * gy
                w01 = gx * fy
                w10 = fx * gy
                w11 = fx * fy
                ws = (w00 * gz, w00 * fz, w01 * gz, w01 * fz,
                      w10 * gz, w10 * fz, w11 * gz, w11 * fz)
                rb = g * (8 * LANES)
                acc0 = None
                acc1 = None
                for c in range(8):
                    f0 = gathv[pl.ds(rb + c * LANES, LANES)]
                    f1 = gathv[pl.ds(HALF + rb + c * LANES, LANES)]
                    if c == 0:
                        acc0 = ws[c] * f0
                        acc1 = ws[c] * f1
                    else:
                        acc0 = acc0 + ws[c] * f0
                        acc1 = acc1 + ws[c] * f1
                outv[pl.ds((2 * lvl) * B + g * LANES, LANES)] = acc0
                outv[pl.ds((2 * lvl + 1) * B + g * LANES, LANES)] = acc1
                return c0

            lax.fori_loop(0, BG, comb_body, 0)

        for r in range(L * F):
            pltpu.sync_copy(outv.at[pl.ds(r * B, B)],
                            out_hbm.at[r, pl.ds(base, B)])
        return carry

    lax.fori_loop(0, NBLK, blk_body, 0)


@jax.jit
def _encode_sc(xs, ys, zs, table):
    mesh = plsc.VectorSubcoreMesh(core_axis_name="c", subcore_axis_name="s")
    return pl.kernel(
        _body,
        out_type=jax.ShapeDtypeStruct((L * F, NUM_POINTS), jnp.float32),
        mesh=mesh,
        scratch_types=[
            pltpu.VMEM((B,), jnp.float32),
            pltpu.VMEM((B,), jnp.float32),
            pltpu.VMEM((B,), jnp.float32),
            pltpu.VMEM((B,), jnp.float32),
            pltpu.VMEM((B,), jnp.float32),
            pltpu.VMEM((B,), jnp.float32),
            pltpu.VMEM((2 * HALF,), jnp.int32),
            pltpu.VMEM((2 * HALF,), jnp.float32),
            pltpu.VMEM((B * L * F,), jnp.float32),
            pltpu.SemaphoreType.DMA,
        ],
    )(xs, ys, zs, table)


def kernel(x, hash_table):
    xs, ys, zs = x[:, 0], x[:, 1], x[:, 2]
    out = _encode_sc(xs, ys, zs, hash_table.reshape(-1))
    return out.T


# speedup vs baseline: 5.2905x; 5.2905x over previous
import functools
import math

import jax
import jax.numpy as jnp
import numpy as np
from jax import lax
from jax.experimental import pallas as pl
from jax.experimental.pallas import tpu as pltpu
from jax.experimental.pallas import tpu_sc as plsc

T = 524288
L = 16
F = 2
N_MIN = 16
N_MAX = 2048
NUM_POINTS = 262144

NC = 2
NS = 16
NW = NC * NS
LANES = 16

PW = NUM_POINTS // NW
B = 512
NBLK = PW // B
BG = B // LANES
ROWS = B * 8

_GROWTH = math.exp((math.log(N_MAX) - math.log(N_MIN)) / (L - 1))
SCALES = [float(math.floor(N_MIN * (_GROWTH ** l))) for l in range(L)]
P2 = -1640531535
P3 = 805459861
MASK = T - 1


def _body(xs_hbm, ys_hbm, zs_hbm, t0_hbm, t1_hbm, out_hbm,
          xv, yv, zv, fxv, fyv, fzv, idxv, g0v, g1v, outv, sem0, sem1):
    wid = lax.axis_index("s") * NC + lax.axis_index("c")

    def blk_body(blk, carry):
        base = wid * PW + blk * B
        pltpu.sync_copy(xs_hbm.at[pl.ds(base, B)], xv)
        pltpu.sync_copy(ys_hbm.at[pl.ds(base, B)], yv)
        pltpu.sync_copy(zs_hbm.at[pl.ds(base, B)], zv)

        for lvl in range(L):
            scale = np.float32(SCALES[lvl])
            lvl_t = np.int32(lvl * T)

            def idx_body(g, c0, lvl_t=lvl_t, scale=scale):
                sl = pl.ds(g * LANES, LANES)
                px = xv[sl] * scale
                py = yv[sl] * scale
                pz = zv[sl] * scale
                ix = px.astype(jnp.int32)
                iy = py.astype(jnp.int32)
                iz = pz.astype(jnp.int32)
                fxv[sl] = px - ix.astype(jnp.float32)
                fyv[sl] = py - iy.astype(jnp.float32)
                fzv[sl] = pz - iz.astype(jnp.float32)
                yp = iy * P2
                zp = iz * P3
                hy1 = yp + P2
                hz1 = zp + P3
                a0 = (ix & MASK) ^ lvl_t
                a1 = ((ix + 1) & MASK) ^ lvl_t
                b00 = (yp ^ zp) & MASK
                b01 = (yp ^ hz1) & MASK
                b10 = (hy1 ^ zp) & MASK
                b11 = (hy1 ^ hz1) & MASK
                rb = g * (8 * LANES)
                idxv[pl.ds(rb + 0 * LANES, LANES)] = a0 ^ b00
                idxv[pl.ds(rb + 1 * LANES, LANES)] = a0 ^ b01
                idxv[pl.ds(rb + 2 * LANES, LANES)] = a0 ^ b10
                idxv[pl.ds(rb + 3 * LANES, LANES)] = a0 ^ b11
                idxv[pl.ds(rb + 4 * LANES, LANES)] = a1 ^ b00
                idxv[pl.ds(rb + 5 * LANES, LANES)] = a1 ^ b01
                idxv[pl.ds(rb + 6 * LANES, LANES)] = a1 ^ b10
                idxv[pl.ds(rb + 7 * LANES, LANES)] = a1 ^ b11
                return c0

            lax.fori_loop(0, BG, idx_body, 0)

            cp0 = pltpu.async_copy(t0_hbm.at[idxv], g0v, sem0)
            cp1 = pltpu.async_copy(t1_hbm.at[idxv], g1v, sem1)
            cp0.wait()
            cp1.wait()

            def comb_body(g, c0, lvl=lvl):
                sl = pl.ds(g * LANES, LANES)
                fx = fxv[sl]
                fy = fyv[sl]
                fz = fzv[sl]
                gx = 1.0 - fx
                gy = 1.0 - fy
                gz = 1.0 - fz
                w00 = gx * gy
                w01 = gx * fy
                w10 = fx * gy
                w11 = fx * fy
                ws = (w00 * gz, w00 * fz, w01 * gz, w01 * fz,
                      w10 * gz, w10 * fz, w11 * gz, w11 * fz)
                rb = g * (8 * LANES)
                acc0 = None
                acc1 = None
                for c in range(8):
                    f0 = g0v[pl.ds(rb + c * LANES, LANES)]
                    f1 = g1v[pl.ds(rb + c * LANES, LANES)]
                    if c == 0:
                        acc0 = ws[c] * f0
                        acc1 = ws[c] * f1
                    else:
                        acc0 = acc0 + ws[c] * f0
                        acc1 = acc1 + ws[c] * f1
                outv[pl.ds((2 * lvl) * B + g * LANES, LANES)] = acc0
                outv[pl.ds((2 * lvl + 1) * B + g * LANES, LANES)] = acc1
                return c0

            lax.fori_loop(0, BG, comb_body, 0)

        for r in range(L * F):
            pltpu.sync_copy(outv.at[pl.ds(r * B, B)],
                            out_hbm.at[r, pl.ds(base, B)])
        return carry

    lax.fori_loop(0, NBLK, blk_body, 0)


@jax.jit
def _encode_sc(xs, ys, zs, t0, t1):
    mesh = plsc.VectorSubcoreMesh(core_axis_name="c", subcore_axis_name="s")
    return pl.kernel(
        _body,
        out_type=jax.ShapeDtypeStruct((L * F, NUM_POINTS), jnp.float32),
        mesh=mesh,
        scratch_types=[
            pltpu.VMEM((B,), jnp.float32),
            pltpu.VMEM((B,), jnp.float32),
            pltpu.VMEM((B,), jnp.float32),
            pltpu.VMEM((B,), jnp.float32),
            pltpu.VMEM((B,), jnp.float32),
            pltpu.VMEM((B,), jnp.float32),
            pltpu.VMEM((ROWS,), jnp.int32),
            pltpu.VMEM((ROWS,), jnp.float32),
            pltpu.VMEM((ROWS,), jnp.float32),
            pltpu.VMEM((B * L * F,), jnp.float32),
            pltpu.SemaphoreType.DMA,
            pltpu.SemaphoreType.DMA,
        ],
    )(xs, ys, zs, t0, t1)


def kernel(x, hash_table):
    xs, ys, zs = x[:, 0], x[:, 1], x[:, 2]
    t0, t1 = hash_table[:, 0], hash_table[:, 1]
    out = _encode_sc(xs, ys, zs, t0, t1)
    return out.T


# speedup vs baseline: 5.3890x; 1.0186x over previous
import functools
import math

import jax
import jax.numpy as jnp
import numpy as np
from jax import lax
from jax.experimental import pallas as pl
from jax.experimental.pallas import tpu as pltpu
from jax.experimental.pallas import tpu_sc as plsc

T = 524288
L = 16
F = 2
N_MIN = 16
N_MAX = 2048
NUM_POINTS = 262144

NC = 2
NS = 16
NW = NC * NS
LANES = 16

PW = NUM_POINTS // NW
B = 512
NBLK = PW // B
BG = B // LANES
ROWS = B * 8

_GROWTH = math.exp((math.log(N_MAX) - math.log(N_MIN)) / (L - 1))
SCALES = [float(math.floor(N_MIN * (_GROWTH ** l))) for l in range(L)]
P2 = -1640531535
P3 = 805459861
MASK = T - 1


def _body(xs_hbm, ys_hbm, zs_hbm, t0_hbm, t1_hbm, out_hbm,
          xv, yv, zv, fx0, fy0, fz0, fx1, fy1, fz1,
          idx0, idx1, g0a, g1a, g0b, g1b, ov2,
          sa0, sa1, sb0, sb1):
    wid = lax.axis_index("s") * NC + lax.axis_index("c")
    wbase = wid * PW
    pltpu.sync_copy(xs_hbm.at[pl.ds(wbase, PW)], xv)
    pltpu.sync_copy(ys_hbm.at[pl.ds(wbase, PW)], yv)
    pltpu.sync_copy(zs_hbm.at[pl.ds(wbase, PW)], zv)

    idxb = (idx0, idx1)
    gb = ((g0a, g1a), (g0b, g1b))
    semb = ((sa0, sa1), (sb0, sb1))
    fracb = ((fx0, fy0, fz0), (fx1, fy1, fz1))

    def make_idx_loop(boff, lvl, p):
        scale = np.float32(SCALES[lvl])
        lvl_t = np.int32(lvl * T)
        idxv = idxb[p]
        fxv, fyv, fzv = fracb[p]

        def idx_body(g, c0):
            sl = pl.ds(g * LANES, LANES)
            xsl = pl.ds(boff + g * LANES, LANES)
            px = xv[xsl] * scale
            py = yv[xsl] * scale
            pz = zv[xsl] * scale
            ix = px.astype(jnp.int32)
            iy = py.astype(jnp.int32)
            iz = pz.astype(jnp.int32)
            fxv[sl] = px - ix.astype(jnp.float32)
            fyv[sl] = py - iy.astype(jnp.float32)
            fzv[sl] = pz - iz.astype(jnp.float32)
            yp = iy * P2
            zp = iz * P3
            hy1 = yp + P2
            hz1 = zp + P3
            a0 = (ix & MASK) ^ lvl_t
            a1 = ((ix + 1) & MASK) ^ lvl_t
            b00 = (yp ^ zp) & MASK
            b01 = (yp ^ hz1) & MASK
            b10 = (hy1 ^ zp) & MASK
            b11 = (hy1 ^ hz1) & MASK
            rb = g * (8 * LANES)
            idxv[pl.ds(rb + 0 * LANES, LANES)] = a0 ^ b00
            idxv[pl.ds(rb + 1 * LANES, LANES)] = a0 ^ b01
            idxv[pl.ds(rb + 2 * LANES, LANES)] = a0 ^ b10
            idxv[pl.ds(rb + 3 * LANES, LANES)] = a0 ^ b11
            idxv[pl.ds(rb + 4 * LANES, LANES)] = a1 ^ b00
            idxv[pl.ds(rb + 5 * LANES, LANES)] = a1 ^ b01
            idxv[pl.ds(rb + 6 * LANES, LANES)] = a1 ^ b10
            idxv[pl.ds(rb + 7 * LANES, LANES)] = a1 ^ b11
            return c0

        lax.fori_loop(0, BG, idx_body, 0)

    def fire(p):
        c0 = pltpu.async_copy(t0_hbm.at[idxb[p]], gb[p][0], semb[p][0])
        c1 = pltpu.async_copy(t1_hbm.at[idxb[p]], gb[p][1], semb[p][1])
        return (c0, c1)

    def make_comb_loop(lvl, p):
        g0v, g1v = gb[p]
        fxv, fyv, fzv = fracb[p]

        def comb_body(g, c0):
            sl = pl.ds(g * LANES, LANES)
            fx = fxv[sl]
            fy = fyv[sl]
            fz = fzv[sl]
            gx = 1.0 - fx
            gy = 1.0 - fy
            gz = 1.0 - fz
            w00 = gx * gy
            w01 = gx * fy
            w10 = fx * gy
            w11 = fx * fy
            ws = (w00 * gz, w00 * fz, w01 * gz, w01 * fz,
                  w10 * gz, w10 * fz, w11 * gz, w11 * fz)
            rb = g * (8 * LANES)
            acc0 = None
            acc1 = None
            for c in range(8):
                f0 = g0v[pl.ds(rb + c * LANES, LANES)]
                f1 = g1v[pl.ds(rb + c * LANES, LANES)]
                if c == 0:
                    acc0 = ws[c] * f0
                    acc1 = ws[c] * f1
                else:
                    acc0 = acc0 + ws[c] * f0
                    acc1 = acc1 + ws[c] * f1
            ov2[2 * lvl, sl] = acc0
            ov2[2 * lvl + 1, sl] = acc1
            return c0

        lax.fori_loop(0, BG, comb_body, 0)

    def blk_body(blk, carry):
        boff = blk * B
        base = wbase + boff
        make_idx_loop(boff, 0, 0)
        cps = fire(0)
        for lvl in range(L):
            p = lvl % 2
            q = 1 - p
            if lvl < L - 1:
                make_idx_loop(boff, lvl + 1, q)
            cps[0].wait()
            cps[1].wait()
            if lvl < L - 1:
                cps = fire(q)
            make_comb_loop(lvl, p)
        pltpu.sync_copy(ov2, out_hbm.at[:, pl.ds(base, B)])
        return carry

    lax.fori_loop(0, NBLK, blk_body, 0)


@jax.jit
def _encode_sc(xs, ys, zs, t0, t1):
    mesh = plsc.VectorSubcoreMesh(core_axis_name="c", subcore_axis_name="s")
    return pl.kernel(
        _body,
        out_type=jax.ShapeDtypeStruct((L * F, NUM_POINTS), jnp.float32),
        mesh=mesh,
        scratch_types=[
            pltpu.VMEM((PW,), jnp.float32),
            pltpu.VMEM((PW,), jnp.float32),
            pltpu.VMEM((PW,), jnp.float32),
            pltpu.VMEM((B,), jnp.float32),
            pltpu.VMEM((B,), jnp.float32),
            pltpu.VMEM((B,), jnp.float32),
            pltpu.VMEM((B,), jnp.float32),
            pltpu.VMEM((B,), jnp.float32),
            pltpu.VMEM((B,), jnp.float32),
            pltpu.VMEM((ROWS,), jnp.int32),
            pltpu.VMEM((ROWS,), jnp.int32),
            pltpu.VMEM((ROWS,), jnp.float32),
            pltpu.VMEM((ROWS,), jnp.float32),
            pltpu.VMEM((ROWS,), jnp.float32),
            pltpu.VMEM((ROWS,), jnp.float32),
            pltpu.VMEM((L * F, B), jnp.float32),
            pltpu.SemaphoreType.DMA,
            pltpu.SemaphoreType.DMA,
            pltpu.SemaphoreType.DMA,
            pltpu.SemaphoreType.DMA,
        ],
    )(xs, ys, zs, t0, t1)


def kernel(x, hash_table):
    xs, ys, zs = x[:, 0], x[:, 1], x[:, 2]
    t0, t1 = hash_table[:, 0], hash_table[:, 1]
    out = _encode_sc(xs, ys, zs, t0, t1)
    return out.T


# speedup vs baseline: 15.5801x; 2.8911x over previous
import functools
import math

import jax
import jax.numpy as jnp
import numpy as np
from jax import lax
from jax.experimental import pallas as pl
from jax.experimental.pallas import tpu as pltpu
from jax.experimental.pallas import tpu_sc as plsc

T = 524288
L = 16
F = 2
N_MIN = 16
N_MAX = 2048
NUM_POINTS = 262144

NC = 2
NS = 16
NW = NC * NS
LANES = 16

PW = NUM_POINTS // NW
B = 512
NBLK = PW // B
BG = B // LANES
ROWS = B * 8
TCHUNK = T // NS

_GROWTH = math.exp((math.log(N_MAX) - math.log(N_MIN)) / (L - 1))
SCALES = [float(math.floor(N_MIN * (_GROWTH ** l))) for l in range(L)]
P2 = -1640531535
P3 = 805459861
MASK = T - 1


def _body(xs_hbm, ys_hbm, zs_hbm, t0_hbm, t1_hbm, sc_hbm, out_hbm,
          xv, yv, zv, fx0, fy0, fz0, fx1, fy1, fz1,
          idx0, idx1, g0a, g1a, g0b, g1b, ova, ovvb, scv, tsh0, tsh1,
          sa0, sa1, sb0, sb1, so0, so1):
    ovb = (ova, ovvb)
    osem = (so0, so1)
    idxb = (idx0, idx1)
    gb = ((g0a, g1a), (g0b, g1b))
    semb = ((sa0, sa1), (sb0, sb1))
    fracb = ((fx0, fy0, fz0), (fx1, fy1, fz1))

    wid = lax.axis_index("s") * NC + lax.axis_index("c")
    sid = lax.axis_index("s")
    wbase = wid * PW
    pltpu.sync_copy(xs_hbm.at[pl.ds(wbase, PW)], xv)
    pltpu.sync_copy(ys_hbm.at[pl.ds(wbase, PW)], yv)
    pltpu.sync_copy(zs_hbm.at[pl.ds(wbase, PW)], zv)
    pltpu.sync_copy(sc_hbm, scv)

    def make_idx_loop(boff, scale, p):
        idxv = idxb[p]
        fxv, fyv, fzv = fracb[p]

        def idx_body(g, c0):
            sl = pl.ds(g * LANES, LANES)
            xsl = pl.ds(boff + g * LANES, LANES)
            px = xv[xsl] * scale
            py = yv[xsl] * scale
            pz = zv[xsl] * scale
            ix = px.astype(jnp.int32)
            iy = py.astype(jnp.int32)
            iz = pz.astype(jnp.int32)
            fxv[sl] = px - ix.astype(jnp.float32)
            fyv[sl] = py - iy.astype(jnp.float32)
            fzv[sl] = pz - iz.astype(jnp.float32)
            yp = iy * P2
            zp = iz * P3
            hy1 = yp + P2
            hz1 = zp + P3
            a0 = ix & MASK
            a1 = (ix + 1) & MASK
            b00 = (yp ^ zp) & MASK
            b01 = (yp ^ hz1) & MASK
            b10 = (hy1 ^ zp) & MASK
            b11 = (hy1 ^ hz1) & MASK
            rb = g * (8 * LANES)
            idxv[pl.ds(rb + 0 * LANES, LANES)] = a0 ^ b00
            idxv[pl.ds(rb + 1 * LANES, LANES)] = a0 ^ b01
            idxv[pl.ds(rb + 2 * LANES, LANES)] = a0 ^ b10
            idxv[pl.ds(rb + 3 * LANES, LANES)] = a0 ^ b11
            idxv[pl.ds(rb + 4 * LANES, LANES)] = a1 ^ b00
            idxv[pl.ds(rb + 5 * LANES, LANES)] = a1 ^ b01
            idxv[pl.ds(rb + 6 * LANES, LANES)] = a1 ^ b10
            idxv[pl.ds(rb + 7 * LANES, LANES)] = a1 ^ b11
            return c0

        lax.fori_loop(0, BG, idx_body, 0)

    def fire(p):
        pltpu.async_copy(tsh0.at[idxb[p]], gb[p][0], semb[p][0])
        pltpu.async_copy(tsh1.at[idxb[p]], gb[p][1], semb[p][1])

    def gwait(p):
        pltpu.make_async_copy(tsh0.at[idxb[p]], gb[p][0], semb[p][0]).wait()
        pltpu.make_async_copy(tsh1.at[idxb[p]], gb[p][1], semb[p][1]).wait()

    def make_comb_loop(p):
        ovl = ovb[p]
        g0v, g1v = gb[p]
        fxv, fyv, fzv = fracb[p]

        def comb_body(g, c0):
            sl = pl.ds(g * LANES, LANES)
            fx = fxv[sl]
            fy = fyv[sl]
            fz = fzv[sl]
            gx = 1.0 - fx
            gy = 1.0 - fy
            gz = 1.0 - fz
            w00 = gx * gy
            w01 = gx * fy
            w10 = fx * gy
            w11 = fx * fy
            ws = (w00 * gz, w00 * fz, w01 * gz, w01 * fz,
                  w10 * gz, w10 * fz, w11 * gz, w11 * fz)
            rb = g * (8 * LANES)
            acc0 = None
            acc1 = None
            for c in range(8):
                f0 = g0v[pl.ds(rb + c * LANES, LANES)]
                f1 = g1v[pl.ds(rb + c * LANES, LANES)]
                if c == 0:
                    acc0 = ws[c] * f0
                    acc1 = ws[c] * f1
                else:
                    acc0 = acc0 + ws[c] * f0
                    acc1 = acc1 + ws[c] * f1
            ovl[0, sl] = acc0
            ovl[1, sl] = acc1
            return c0

        lax.fori_loop(0, BG, comb_body, 0)

    def ofire(row0, blk, p):
        obase = row0 + wbase + blk * B
        pltpu.async_copy(ovb[p].at[0], out_hbm.at[pl.ds(obase, B)], osem[p])
        pltpu.async_copy(
            ovb[p].at[1], out_hbm.at[pl.ds(obase + NUM_POINTS, B)], osem[p])

    def owait(p):
        pltpu.make_async_copy(
            ovb[p].at[0], out_hbm.at[pl.ds(0, B)], osem[p]).wait()
        pltpu.make_async_copy(
            ovb[p].at[1], out_hbm.at[pl.ds(0, B)], osem[p]).wait()

    def lvl_body(lvl, carry):
        cbase = lvl * T + sid * TCHUNK
        pltpu.sync_copy(t0_hbm.at[pl.ds(cbase, TCHUNK)],
                        tsh0.at[pl.ds(sid * TCHUNK, TCHUNK)])
        pltpu.sync_copy(t1_hbm.at[pl.ds(cbase, TCHUNK)],
                        tsh1.at[pl.ds(sid * TCHUNK, TCHUNK)])
        plsc.subcore_barrier()

        scale = scv[pl.ds(lvl * LANES, LANES)]
        row0 = 2 * lvl * NUM_POINTS

        make_idx_loop(0, scale, 0)
        fire(0)
        make_idx_loop(B, scale, 1)
        gwait(0)
        fire(1)
        make_comb_loop(0)
        ofire(row0, 0, 0)
        make_idx_loop(2 * B, scale, 0)
        gwait(1)
        fire(0)
        make_comb_loop(1)
        ofire(row0, 1, 1)

        def blk2_body(i, c0):
            b0 = 2 * i
            make_idx_loop((b0 + 1) * B, scale, 1)
            gwait(0)
            fire(1)
            owait(0)
            make_comb_loop(0)
            ofire(row0, b0, 0)
            make_idx_loop((b0 + 2) * B, scale, 0)
            gwait(1)
            fire(0)
            owait(1)
            make_comb_loop(1)
            ofire(row0, b0 + 1, 1)
            return c0

        lax.fori_loop(1, NBLK // 2 - 1, blk2_body, 0)

        make_idx_loop(15 * B, scale, 1)
        gwait(0)
        fire(1)
        owait(0)
        make_comb_loop(0)
        ofire(row0, 14, 0)
        gwait(1)
        owait(1)
        make_comb_loop(1)
        ofire(row0, 15, 1)
        owait(0)
        owait(1)
        plsc.subcore_barrier()
        return carry

    lax.fori_loop(0, L, lvl_body, 0)


@jax.jit
def _encode_sc(xs, ys, zs, t0, t1, scales):
    mesh = plsc.VectorSubcoreMesh(core_axis_name="c", subcore_axis_name="s")
    return pl.kernel(
        _body,
        out_type=jax.ShapeDtypeStruct((L * F * NUM_POINTS,), jnp.float32),
        mesh=mesh,
        scratch_types=[
            pltpu.VMEM((PW,), jnp.float32),
            pltpu.VMEM((PW,), jnp.float32),
            pltpu.VMEM((PW,), jnp.float32),
            pltpu.VMEM((B,), jnp.float32),
            pltpu.VMEM((B,), jnp.float32),
            pltpu.VMEM((B,), jnp.float32),
            pltpu.VMEM((B,), jnp.float32),
            pltpu.VMEM((B,), jnp.float32),
            pltpu.VMEM((B,), jnp.float32),
            pltpu.VMEM((ROWS,), jnp.int32),
            pltpu.VMEM((ROWS,), jnp.int32),
            pltpu.VMEM((ROWS,), jnp.float32),
            pltpu.VMEM((ROWS,), jnp.float32),
            pltpu.VMEM((ROWS,), jnp.float32),
            pltpu.VMEM((ROWS,), jnp.float32),
            pltpu.VMEM((F, B), jnp.float32),
            pltpu.VMEM((F, B), jnp.float32),
            pltpu.VMEM((L * LANES,), jnp.float32),
            pltpu.VMEM_SHARED((T,), jnp.float32),
            pltpu.VMEM_SHARED((T,), jnp.float32),
            pltpu.SemaphoreType.DMA,
            pltpu.SemaphoreType.DMA,
            pltpu.SemaphoreType.DMA,
            pltpu.SemaphoreType.DMA,
            pltpu.SemaphoreType.DMA,
            pltpu.SemaphoreType.DMA,
        ],
    )(xs, ys, zs, t0, t1, scales)


def kernel(x, hash_table):
    xs, ys, zs = x[:, 0], x[:, 1], x[:, 2]
    t0, t1 = hash_table[:, 0], hash_table[:, 1]
    scales = jnp.asarray(
        np.repeat(np.array(SCALES, dtype=np.float32), LANES))
    out = _encode_sc(xs, ys, zs, t0, t1, scales)
    return out.reshape(L * F, NUM_POINTS).T


# speedup vs baseline: 15.6906x; 1.0071x over previous
import functools
import math

import jax
import jax.numpy as jnp
import numpy as np
from jax import lax
from jax.experimental import pallas as pl
from jax.experimental.pallas import tpu as pltpu
from jax.experimental.pallas import tpu_sc as plsc

T = 524288
L = 16
F = 2
N_MIN = 16
N_MAX = 2048
NUM_POINTS = 262144

NC = 2
NS = 16
NW = NC * NS
LANES = 16

PW = NUM_POINTS // NW
B = 512
NBLK = PW // B
BG = B // LANES
ROWS = B * 8
TCHUNK = T // NS

_GROWTH = math.exp((math.log(N_MAX) - math.log(N_MIN)) / (L - 1))
SCALES = [float(math.floor(N_MIN * (_GROWTH ** l))) for l in range(L)]
P2 = -1640531535
P3 = 805459861
MASK = T - 1


def _body(xs_hbm, ys_hbm, zs_hbm, t0_hbm, t1_hbm, sc_hbm, out_hbm,
          xv, yv, zv, fx0, fy0, fz0, fx1, fy1, fz1,
          idx0, idx1, g0a, g1a, g0b, g1b, ova, ovvb, scv, tsh0, tsh1,
          sa0, sa1, sb0, sb1, so0, so1):
    ovb = (ova, ovvb)
    osem = (so0, so1)
    idxb = (idx0, idx1)
    gb = ((g0a, g1a), (g0b, g1b))
    semb = ((sa0, sa1), (sb0, sb1))
    fracb = ((fx0, fy0, fz0), (fx1, fy1, fz1))

    wid = lax.axis_index("s") * NC + lax.axis_index("c")
    sid = lax.axis_index("s")
    wbase = wid * PW
    pltpu.sync_copy(xs_hbm.at[pl.ds(wbase, PW)], xv)
    pltpu.sync_copy(ys_hbm.at[pl.ds(wbase, PW)], yv)
    pltpu.sync_copy(zs_hbm.at[pl.ds(wbase, PW)], zv)
    pltpu.sync_copy(sc_hbm, scv)

    def make_idx_loop(boff, scale, p):
        idxv = idxb[p]
        fxv, fyv, fzv = fracb[p]

        @plsc.parallel_loop(0, BG, unroll=2)
        def idx_body(g):
            sl = pl.ds(g * LANES, LANES)
            xsl = pl.ds(boff + g * LANES, LANES)
            px = xv[xsl] * scale
            py = yv[xsl] * scale
            pz = zv[xsl] * scale
            ix = px.astype(jnp.int32)
            iy = py.astype(jnp.int32)
            iz = pz.astype(jnp.int32)
            fxv[sl] = px - ix.astype(jnp.float32)
            fyv[sl] = py - iy.astype(jnp.float32)
            fzv[sl] = pz - iz.astype(jnp.float32)
            yp = iy * P2
            zp = iz * P3
            hy1 = yp + P2
            hz1 = zp + P3
            a0 = ix & MASK
            a1 = (ix + 1) & MASK
            b00 = (yp ^ zp) & MASK
            b01 = (yp ^ hz1) & MASK
            b10 = (hy1 ^ zp) & MASK
            b11 = (hy1 ^ hz1) & MASK
            rb = g * (8 * LANES)
            idxv[pl.ds(rb + 0 * LANES, LANES)] = a0 ^ b00
            idxv[pl.ds(rb + 1 * LANES, LANES)] = a0 ^ b01
            idxv[pl.ds(rb + 2 * LANES, LANES)] = a0 ^ b10
            idxv[pl.ds(rb + 3 * LANES, LANES)] = a0 ^ b11
            idxv[pl.ds(rb + 4 * LANES, LANES)] = a1 ^ b00
            idxv[pl.ds(rb + 5 * LANES, LANES)] = a1 ^ b01
            idxv[pl.ds(rb + 6 * LANES, LANES)] = a1 ^ b10
            idxv[pl.ds(rb + 7 * LANES, LANES)] = a1 ^ b11

    def fire(p):
        pltpu.async_copy(tsh0.at[idxb[p]], gb[p][0], semb[p][0])
        pltpu.async_copy(tsh1.at[idxb[p]], gb[p][1], semb[p][1])

    def gwait(p):
        pltpu.make_async_copy(tsh0.at[idxb[p]], gb[p][0], semb[p][0]).wait()
        pltpu.make_async_copy(tsh1.at[idxb[p]], gb[p][1], semb[p][1]).wait()

    def make_comb_loop(p):
        ovl = ovb[p]
        g0v, g1v = gb[p]
        fxv, fyv, fzv = fracb[p]

        @plsc.parallel_loop(0, BG, unroll=2)
        def comb_body(g):
            sl = pl.ds(g * LANES, LANES)
            fx = fxv[sl]
            fy = fyv[sl]
            fz = fzv[sl]
            gx = 1.0 - fx
            gy = 1.0 - fy
            gz = 1.0 - fz
            w00 = gx * gy
            w01 = gx * fy
            w10 = fx * gy
            w11 = fx * fy
            ws = (w00 * gz, w00 * fz, w01 * gz, w01 * fz,
                  w10 * gz, w10 * fz, w11 * gz, w11 * fz)
            rb = g * (8 * LANES)
            acc0 = None
            acc1 = None
            for c in range(8):
                f0 = g0v[pl.ds(rb + c * LANES, LANES)]
                f1 = g1v[pl.ds(rb + c * LANES, LANES)]
                if c == 0:
                    acc0 = ws[c] * f0
                    acc1 = ws[c] * f1
                else:
                    acc0 = acc0 + ws[c] * f0
                    acc1 = acc1 + ws[c] * f1
            ovl[0, sl] = acc0
            ovl[1, sl] = acc1

    def ofire(row0, blk, p):
        obase = row0 + wbase + blk * B
        pltpu.async_copy(ovb[p].at[0], out_hbm.at[pl.ds(obase, B)], osem[p])
        pltpu.async_copy(
            ovb[p].at[1], out_hbm.at[pl.ds(obase + NUM_POINTS, B)], osem[p])

    def owait(p):
        pltpu.make_async_copy(
            ovb[p].at[0], out_hbm.at[pl.ds(0, B)], osem[p]).wait()
        pltpu.make_async_copy(
            ovb[p].at[1], out_hbm.at[pl.ds(0, B)], osem[p]).wait()

    def lvl_body(lvl, carry):
        cbase = lvl * T + sid * TCHUNK
        pltpu.sync_copy(t0_hbm.at[pl.ds(cbase, TCHUNK)],
                        tsh0.at[pl.ds(sid * TCHUNK, TCHUNK)])
        pltpu.sync_copy(t1_hbm.at[pl.ds(cbase, TCHUNK)],
                        tsh1.at[pl.ds(sid * TCHUNK, TCHUNK)])
        plsc.subcore_barrier()

        scale = scv[pl.ds(lvl * LANES, LANES)]
        row0 = 2 * lvl * NUM_POINTS

        make_idx_loop(0, scale, 0)
        fire(0)
        make_idx_loop(B, scale, 1)
        gwait(0)
        fire(1)
        make_comb_loop(0)
        ofire(row0, 0, 0)
        make_idx_loop(2 * B, scale, 0)
        gwait(1)
        fire(0)
        make_comb_loop(1)
        ofire(row0, 1, 1)

        def blk2_body(i, c0):
            b0 = 2 * i
            make_idx_loop((b0 + 1) * B, scale, 1)
            gwait(0)
            fire(1)
            owait(0)
            make_comb_loop(0)
            ofire(row0, b0, 0)
            make_idx_loop((b0 + 2) * B, scale, 0)
            gwait(1)
            fire(0)
            owait(1)
            make_comb_loop(1)
            ofire(row0, b0 + 1, 1)
            return c0

        lax.fori_loop(1, NBLK // 2 - 1, blk2_body, 0)

        make_idx_loop(15 * B, scale, 1)
        gwait(0)
        fire(1)
        owait(0)
        make_comb_loop(0)
        ofire(row0, 14, 0)
        gwait(1)
        owait(1)
        make_comb_loop(1)
        ofire(row0, 15, 1)
        owait(0)
        owait(1)
        plsc.subcore_barrier()
        return carry

    lax.fori_loop(0, L, lvl_body, 0)


@jax.jit
def _encode_sc(xs, ys, zs, t0, t1, scales):
    mesh = plsc.VectorSubcoreMesh(core_axis_name="c", subcore_axis_name="s")
    return pl.kernel(
        _body,
        out_type=jax.ShapeDtypeStruct((L * F * NUM_POINTS,), jnp.float32),
        mesh=mesh,
        scratch_types=[
            pltpu.VMEM((PW,), jnp.float32),
            pltpu.VMEM((PW,), jnp.float32),
            pltpu.VMEM((PW,), jnp.float32),
            pltpu.VMEM((B,), jnp.float32),
            pltpu.VMEM((B,), jnp.float32),
            pltpu.VMEM((B,), jnp.float32),
            pltpu.VMEM((B,), jnp.float32),
            pltpu.VMEM((B,), jnp.float32),
            pltpu.VMEM((B,), jnp.float32),
            pltpu.VMEM((ROWS,), jnp.int32),
            pltpu.VMEM((ROWS,), jnp.int32),
            pltpu.VMEM((ROWS,), jnp.float32),
            pltpu.VMEM((ROWS,), jnp.float32),
            pltpu.VMEM((ROWS,), jnp.float32),
            pltpu.VMEM((ROWS,), jnp.float32),
            pltpu.VMEM((F, B), jnp.float32),
            pltpu.VMEM((F, B), jnp.float32),
            pltpu.VMEM((L * LANES,), jnp.float32),
            pltpu.VMEM_SHARED((T,), jnp.float32),
            pltpu.VMEM_SHARED((T,), jnp.float32),
            pltpu.SemaphoreType.DMA,
            pltpu.SemaphoreType.DMA,
            pltpu.SemaphoreType.DMA,
            pltpu.SemaphoreType.DMA,
            pltpu.SemaphoreType.DMA,
            pltpu.SemaphoreType.DMA,
        ],
    )(xs, ys, zs, t0, t1, scales)


def kernel(x, hash_table):
    xs, ys, zs = x[:, 0], x[:, 1], x[:, 2]
    t0, t1 = hash_table[:, 0], hash_table[:, 1]
    scales = jnp.asarray(
        np.repeat(np.array(SCALES, dtype=np.float32), LANES))
    out = _encode_sc(xs, ys, zs, t0, t1, scales)
    return out.reshape(L * F, NUM_POINTS).T
